# baseline (speedup 1.0000x reference)
import functools

import jax
import jax.numpy as jnp
from jax import lax
from jax.experimental import pallas as pl
from jax.experimental.pallas import tpu as pltpu
from jax.experimental.pallas import tpu_sc as plsc

N = 8192
NI = 4096
F = 64
BLK = 256
NB = N // BLK
ALPHA = 0.1

_NS = 16
_L = 16
_ROWS = N // 128
_RPS = _ROWS // _NS


def _sc_body(uim_flat, colidx, uservec, x_out,
             idxv, gatv, xv, uv, sem):
    cid = lax.axis_index("c")
    sid = lax.axis_index("s")

    @pl.when(cid == 0)
    def _column_branch():
        pltpu.sync_copy(colidx, idxv)
        descs = [pltpu.async_copy(uim_flat.at[idxv.at[j]], gatv.at[j], sem)
                 for j in range(_ROWS)]
        for d in descs:
            d.wait()

        tot = jnp.zeros((_L,), jnp.float32)
        for j in range(_ROWS):
            for c in range(128 // _L):
                tot = tot + gatv[j, pl.ds(c * _L, _L)]
        s = 0.0
        for l in range(_L):
            s = s + tot[l]
        mean = s * (1.0 / N)
        mean16 = jnp.full((_L,), mean, jnp.float32)

        pltpu.sync_copy(uservec, uv)
        user16 = uv[...]
        for j in range(_RPS):
            gj = sid * _RPS + j
            base = gj * 128
            for c in range(128 // _L):
                pos = jnp.full((_L,), base + c * _L, jnp.int32) + \
                    lax.iota(jnp.int32, _L)
                v = gatv[gj, pl.ds(c * _L, _L)] - mean16
                v = jnp.where(pos == user16, 0.0, v)
                xv[j, pl.ds(c * _L, _L)] = v
        pltpu.sync_copy(xv, x_out.at[pl.ds(sid * _RPS, _RPS)])


def _sc_prologue(uim_flat, colidx, uservec):
    f32 = jnp.float32
    kern = pl.kernel(
        _sc_body,
        mesh=plsc.VectorSubcoreMesh(core_axis_name="c", subcore_axis_name="s"),
        out_type=[
            jax.ShapeDtypeStruct((_ROWS, 128), f32),
        ],
        scratch_types=[
            pltpu.VMEM((_ROWS, 128), jnp.int32),
            pltpu.VMEM((_ROWS, 128), f32),
            pltpu.VMEM((_RPS, 128), f32),
            pltpu.VMEM((_L,), jnp.int32),
            pltpu.SemaphoreType.DMA,
        ],
    )
    return kern(uim_flat, colidx, uservec)


def _b16(a):
    b = jax.lax.bitcast_convert_type(a, jnp.int32)
    r = b + jnp.int32(0x7FFF) + ((b >> 16) & jnp.int32(1))
    r = r & jnp.int32(-65536)
    return jax.lax.bitcast_convert_type(r, jnp.float32)


def _tc_body(s_ref, s_sim_ref, s_dis_ref, usim_band, udis_band, x_col,
             fsim_ref, fdis_ref, ssim_ref, sdis_ref, h_ref,
             out_ref, zs_acc, zd_acc):
    i = pl.program_id(0)

    dn_mm = (((1,), (0,)), ((), ()))
    f32 = jnp.float32
    zs_acc[pl.ds(i * BLK, BLK), :] = jax.lax.dot_general(
        s_sim_ref[...], x_col[...], dn_mm, preferred_element_type=f32)
    zd_acc[pl.ds(i * BLK, BLK), :] = jax.lax.dot_general(
        s_dis_ref[...], x_col[...], dn_mm, preferred_element_type=f32)

    @pl.when(i == NB - 1)
    def _fin():
        r = s_ref[1]
        usim_b = usim_band[pl.ds(r, 1), :]
        udis_b = udis_band[pl.ds(r, 1), :]
        t1s = jax.lax.dot_general(usim_b, x_col[...], dn_mm,
                                  preferred_element_type=f32)[0, 0]
        t1d = jax.lax.dot_general(udis_b, x_col[...], dn_mm,
                                  preferred_element_type=f32)[0, 0]
        t2s = jax.lax.dot_general(usim_b, zs_acc[...], dn_mm,
                                  preferred_element_type=f32)[0, 0]
        t2d = jax.lax.dot_general(udis_b, zd_acc[...], dn_mm,
                                  preferred_element_type=f32)[0, 0]
        ys = t1s * h_ref[0:1, :] + t2s * h_ref[1:2, :]
        yd = t1d * h_ref[2:3, :] + t2d * h_ref[3:4, :]
        ys = jnp.where(ys >= 0.0, ys, 0.01 * ys)
        yd = jnp.where(yd >= 0.0, yd, 0.01 * yd)
        qs = jax.lax.dot_general(ys, fsim_ref[...], dn_mm,
                                 preferred_element_type=f32)
        qd = jax.lax.dot_general(yd, fdis_ref[...], dn_mm,
                                 preferred_element_type=f32)
        vs = jax.lax.dot_general(qs, ssim_ref[...], dn_mm,
                                 preferred_element_type=f32)[0, 0]
        vd = jax.lax.dot_general(qd, sdis_ref[...], dn_mm,
                                 preferred_element_type=f32)[0, 0]
        out_ref[...] = jnp.full((1, 1), (1.0 - ALPHA) * vs + ALPHA * vd,
                                jnp.float32)


def _tc_core(scalars, S_sim, S_dis, x_col, fsim, fdis,
             ssim_col, sdis_col, h_pack, interpret=False):
    grid_spec = pltpu.PrefetchScalarGridSpec(
        num_scalar_prefetch=1,
        grid=(NB,),
        in_specs=[
            pl.BlockSpec((BLK, N), lambda i, s: (i, 0)),
            pl.BlockSpec((BLK, N), lambda i, s: (i, 0)),
            pl.BlockSpec((8, N), lambda i, s: (s[0], 0)),
            pl.BlockSpec((8, N), lambda i, s: (s[0], 0)),
            pl.BlockSpec((N, 1), lambda i, s: (0, 0)),
            pl.BlockSpec((F, N), lambda i, s: (0, 0)),
            pl.BlockSpec((F, N), lambda i, s: (0, 0)),
            pl.BlockSpec((N, 1), lambda i, s: (0, 0)),
            pl.BlockSpec((N, 1), lambda i, s: (0, 0)),
            pl.BlockSpec((8, F), lambda i, s: (0, 0)),
        ],
        out_specs=pl.BlockSpec((1, 1), lambda i, s: (0, 0)),
        scratch_shapes=[
            pltpu.VMEM((N, 1), jnp.float32),
            pltpu.VMEM((N, 1), jnp.float32),
        ],
    )
    return pl.pallas_call(
        _tc_body,
        grid_spec=grid_spec,
        out_shape=jax.ShapeDtypeStruct((1, 1), jnp.float32),
        compiler_params=pltpu.CompilerParams(
            dimension_semantics=("arbitrary",),
        ),
        interpret=interpret,
    )(scalars, S_sim, S_dis, S_sim, S_dis, x_col, fsim, fdis,
      ssim_col, sdis_col, h_pack)


def kernel(user, item, uimTrain, S_sim, S_dis, hSim, hDis,
           finalSimFC, sharedSimFC, finalDisFC, sharedDisFC):
    user = jnp.asarray(user, jnp.int32)
    item = jnp.asarray(item, jnp.int32)

    uim_flat = uimTrain.reshape(N * NI)
    colidx = (jnp.arange(N, dtype=jnp.int32) * NI + item).reshape(_ROWS, 128)
    uservec = jnp.full((_L,), user, jnp.int32)
    (x2d,) = _sc_prologue(uim_flat, colidx, uservec)
    x_col = x2d.reshape(N, 1)

    scalars = jnp.stack([user // 8, user % 8]).astype(jnp.int32)
    h_pack = jnp.zeros((8, F), jnp.float32)
    h_pack = h_pack.at[0, :].set(hSim[1, 0])
    h_pack = h_pack.at[1, :].set(hSim[2, 0])
    h_pack = h_pack.at[2, :].set(hDis[1, 0])
    h_pack = h_pack.at[3, :].set(hDis[2, 0])

    out = _tc_core(scalars, S_sim, S_dis, x_col,
                   finalSimFC, finalDisFC, sharedSimFC, sharedDisFC, h_pack)
    return out.reshape(1)
